# baseline (speedup 1.0000x reference)
import functools

import jax
import jax.numpy as jnp
from jax import lax
from jax.experimental import pallas as pl
from jax.experimental.pallas import tpu as pltpu
from jax.experimental.pallas import tpu_sc as plsc

E = 8
H = 768
F = 3072
T = 2048

BM = 256
NB = T // BM
NUM_TILES = NB + E - 1

NC = 2
NS = 16
NW = NC * NS
BPW = T // NW


def _cumsum(a, axis):
    n = a.shape[axis]
    k = 1
    while k < n:
        zeros = jnp.zeros_like(lax.slice_in_dim(a, 0, k, axis=axis))
        shifted = jnp.concatenate(
            [zeros, lax.slice_in_dim(a, 0, n - k, axis=axis)], axis=axis)
        a = a + shifted
        k *= 2
    return a


def _routing_body(idx_ref, pos_ref, tiles_ref):
    idx = idx_ref[...]
    lane_e = lax.broadcasted_iota(jnp.int32, (T, E), 1)
    e1h = (idx == lane_e).astype(jnp.int32)
    csum = _cumsum(e1h, axis=0)
    counts = csum[T - 1 : T, :]
    offs_incl = _cumsum(counts, axis=1)
    offs_excl = offs_incl - counts

    pos = jnp.sum(e1h * (offs_excl + csum - 1), axis=1, keepdims=True)
    pos_ref[...] = pos

    lo = offs_excl // BM
    hi = (offs_incl - 1) // BM
    nt = jnp.where(counts > 0, hi - lo + 1, 0)
    t_incl = _cumsum(nt, axis=1)
    t_excl = t_incl - nt
    v = t_incl[0, E - 1]

    jj = lax.broadcasted_iota(jnp.int32, (NUM_TILES, 1), 0)
    jc = jnp.minimum(jj, v - 1)
    sel = ((jc >= t_excl) & (jc < t_incl)).astype(jnp.int32)

    def pick(val):
        return jnp.sum(sel * val, axis=1, keepdims=True)

    ex = pick(lax.broadcasted_iota(jnp.int32, (NUM_TILES, E), 1))
    blk = pick(lo) + (jc - pick(t_excl))
    start = jnp.maximum(pick(offs_excl), blk * BM)
    end = jnp.minimum(pick(offs_incl), (blk + 1) * BM)
    start = jnp.where(jj > jc, end, start)

    nonz = (counts > 0).astype(jnp.int32)
    ord_incl = _cumsum(nonz, axis=1)
    slot_e = (ord_incl - nonz) % 2
    eid = lax.broadcasted_iota(jnp.int32, (1, E), 1)
    vals = jnp.where(counts > 0, eid, E)
    nxt = jnp.full_like(vals, E)
    k = 1
    while k < E:
        pad = jnp.full_like(lax.slice_in_dim(vals, 0, k, axis=1), E)
        shifted = jnp.concatenate(
            [lax.slice_in_dim(vals, k, E, axis=1), pad], axis=1)
        nxt = jnp.minimum(nxt, shifted)
        vals = jnp.minimum(vals, shifted)
        k *= 2
    slot = pick(slot_e)
    first = ((jc == pick(t_excl)) & (jj == jc)).astype(jnp.int32)
    next_ex = pick(nxt)
    has_next = (next_ex < E).astype(jnp.int32)
    next_ex = jnp.minimum(next_ex, E - 1)
    tiles_ref[...] = jnp.concatenate(
        [blk, ex, start, end, slot, first, next_ex, has_next], axis=1)


def _routing(idx2):
    return pl.pallas_call(
        _routing_body,
        out_shape=(
            jax.ShapeDtypeStruct((T, 1), jnp.int32),
            jax.ShapeDtypeStruct((NUM_TILES, 8), jnp.int32),
        ),
    )(idx2)


@functools.cache
def _sc_mesh():
    return plsc.VectorSubcoreMesh(core_axis_name="c", subcore_axis_name="s")


def _dispatch(x, pos_w):
    @functools.partial(
        pl.kernel,
        mesh=_sc_mesh(),
        out_type=jax.ShapeDtypeStruct((T, H), jnp.float32),
        scratch_types=[
            pltpu.VMEM((BPW,), jnp.int32),
            pltpu.VMEM((BPW, H), jnp.float32),
            pltpu.SemaphoreType.DMA,
        ],
    )
    def k(x_hbm, pos_hbm, o_hbm, idx_v, rows_v, sem):
        wid = lax.axis_index("s") * NC + lax.axis_index("c")
        base = wid * BPW
        pltpu.sync_copy(pos_hbm.at[wid], idx_v)
        pltpu.sync_copy(x_hbm.at[pl.ds(base, BPW)], rows_v)
        pltpu.async_copy(rows_v, o_hbm.at[idx_v], sem).wait()

    return k(x, pos_w)


def _combine(y_sorted, pos_w):
    @functools.partial(
        pl.kernel,
        mesh=_sc_mesh(),
        out_type=jax.ShapeDtypeStruct((T, H), jnp.float32),
        scratch_types=[
            pltpu.VMEM((BPW,), jnp.int32),
            pltpu.VMEM((BPW, H), jnp.float32),
            pltpu.SemaphoreType.DMA,
        ],
    )
    def k(y_hbm, pos_hbm, o_hbm, idx_v, rows_v, sem):
        wid = lax.axis_index("s") * NC + lax.axis_index("c")
        base = wid * BPW
        pltpu.sync_copy(pos_hbm.at[wid], idx_v)
        pltpu.async_copy(y_hbm.at[idx_v], rows_v, sem).wait()
        pltpu.sync_copy(rows_v, o_hbm.at[pl.ds(base, BPW)])

    return k(y_sorted, pos_w)


def _moe_body(tiles_ref, x_ref, w1_hbm, b1_ref, w2_hbm, b2_ref, o_ref,
              w1_buf, w2_buf, s1, s2):
    j = pl.program_id(0)
    blk = tiles_ref[j, 0]
    ex = tiles_ref[j, 1]
    start = tiles_ref[j, 2]
    end = tiles_ref[j, 3]
    slot = tiles_ref[j, 4]
    first = tiles_ref[j, 5]
    nxt = tiles_ref[j, 6]
    has_next = tiles_ref[j, 7]

    @pl.when(j == 0)
    def _():
        pltpu.make_async_copy(w1_hbm.at[ex], w1_buf.at[slot], s1).start()
        pltpu.make_async_copy(w2_hbm.at[ex], w2_buf.at[slot], s2).start()

    @pl.when(first == 1)
    def _():
        pltpu.make_async_copy(w1_hbm.at[ex], w1_buf.at[slot], s1).wait()

        @pl.when(has_next == 1)
        def _():
            pltpu.make_async_copy(
                w1_hbm.at[nxt], w1_buf.at[1 - slot], s1).start()

    h = jnp.dot(x_ref[...], w1_buf[slot], precision=lax.Precision.DEFAULT,
                preferred_element_type=jnp.float32)
    h = h + b1_ref[pl.ds(ex, 1), :]
    h = 0.5 * h * (1.0 + lax.erf(h * 0.7071067811865476))

    @pl.when(first == 1)
    def _():
        pltpu.make_async_copy(w2_hbm.at[ex], w2_buf.at[slot], s2).wait()

        @pl.when(has_next == 1)
        def _():
            pltpu.make_async_copy(
                w2_hbm.at[nxt], w2_buf.at[1 - slot], s2).start()

    y = jnp.dot(h, w2_buf[slot], precision=lax.Precision.DEFAULT,
                preferred_element_type=jnp.float32)
    y = y + b2_ref[pl.ds(ex, 1), :]

    rows = blk * BM + lax.broadcasted_iota(jnp.int32, (BM, 1), 0)
    contrib = jnp.where((rows >= start) & (rows < end), y, 0.0)

    @pl.when(start == blk * BM)
    def _():
        o_ref[...] = contrib

    @pl.when(start != blk * BM)
    def _():
        o_ref[...] += contrib


def _moe(tiles, x_sorted, w1, b1, w2, b2):
    grid_spec = pltpu.PrefetchScalarGridSpec(
        num_scalar_prefetch=1,
        grid=(NUM_TILES,),
        in_specs=[
            pl.BlockSpec((BM, H), lambda j, t: (t[j, 0], 0)),
            pl.BlockSpec(memory_space=pl.ANY),
            pl.BlockSpec((E, F), lambda j, t: (0, 0)),
            pl.BlockSpec(memory_space=pl.ANY),
            pl.BlockSpec((E, H), lambda j, t: (0, 0)),
        ],
        out_specs=pl.BlockSpec((BM, H), lambda j, t: (t[j, 0], 0)),
        scratch_shapes=[
            pltpu.VMEM((2, H, F), jnp.float32),
            pltpu.VMEM((2, F, H), jnp.float32),
            pltpu.SemaphoreType.DMA,
            pltpu.SemaphoreType.DMA,
        ],
    )
    return pl.pallas_call(
        _moe_body,
        grid_spec=grid_spec,
        out_shape=jax.ShapeDtypeStruct((T, H), jnp.float32),
    )(tiles, x_sorted, w1, b1, w2, b2)


def kernel(x, expert_indices, W1, b1, W2, b2):
    idx2 = expert_indices.reshape(T, 1).astype(jnp.int32)
    pos, tiles = _routing(idx2)
    pos_w = pos.reshape(NW, BPW)
    x_sorted = _dispatch(x, pos_w)
    y_sorted = _moe(tiles, x_sorted, W1, b1, W2, b2)
    return _combine(y_sorted, pos_w)


# speedup vs baseline: 1.1275x; 1.1275x over previous
import functools

import jax
import jax.numpy as jnp
from jax import lax
from jax.experimental import pallas as pl
from jax.experimental.pallas import tpu as pltpu
from jax.experimental.pallas import tpu_sc as plsc

E = 8
H = 768
F = 3072
T = 2048

BM = 256
NB = T // BM
NUM_TILES = NB + E - 1

NC = 2
NS = 16
NW = NC * NS
BPW = T // NW


def _cumsum(a, axis):
    n = a.shape[axis]
    k = 1
    while k < n:
        zeros = jnp.zeros_like(lax.slice_in_dim(a, 0, k, axis=axis))
        shifted = jnp.concatenate(
            [zeros, lax.slice_in_dim(a, 0, n - k, axis=axis)], axis=axis)
        a = a + shifted
        k *= 2
    return a


def _routing_body(idx_ref, pos_ref, tiles_ref):
    idx = idx_ref[...]
    lane_e = lax.broadcasted_iota(jnp.int32, (T, E), 1)
    e1h = (idx == lane_e).astype(jnp.int32)
    csum = _cumsum(e1h, axis=0)
    counts = csum[T - 1 : T, :]
    offs_incl = _cumsum(counts, axis=1)
    offs_excl = offs_incl - counts

    pos = jnp.sum(e1h * (offs_excl + csum - 1), axis=1, keepdims=True)
    pos_ref[...] = pos

    lo = offs_excl // BM
    hi = (offs_incl - 1) // BM
    nt = jnp.where(counts > 0, hi - lo + 1, 0)
    t_incl = _cumsum(nt, axis=1)
    t_excl = t_incl - nt
    v = t_incl[0, E - 1]

    jj = lax.broadcasted_iota(jnp.int32, (NUM_TILES, 1), 0)
    jc = jnp.minimum(jj, v - 1)
    sel = ((jc >= t_excl) & (jc < t_incl)).astype(jnp.int32)

    def pick(val):
        return jnp.sum(sel * val, axis=1, keepdims=True)

    ex = pick(lax.broadcasted_iota(jnp.int32, (NUM_TILES, E), 1))
    blk = pick(lo) + (jc - pick(t_excl))
    start = jnp.maximum(pick(offs_excl), blk * BM)
    end = jnp.minimum(pick(offs_incl), (blk + 1) * BM)
    start = jnp.where(jj > jc, end, start)

    nonz = (counts > 0).astype(jnp.int32)
    ord_incl = _cumsum(nonz, axis=1)
    slot_e = (ord_incl - nonz) % 2
    eid = lax.broadcasted_iota(jnp.int32, (1, E), 1)
    vals = jnp.where(counts > 0, eid, E)
    nxt = jnp.full_like(vals, E)
    k = 1
    while k < E:
        pad = jnp.full_like(lax.slice_in_dim(vals, 0, k, axis=1), E)
        shifted = jnp.concatenate(
            [lax.slice_in_dim(vals, k, E, axis=1), pad], axis=1)
        nxt = jnp.minimum(nxt, shifted)
        vals = jnp.minimum(vals, shifted)
        k *= 2
    nxt2 = jnp.full_like(nxt, E)
    for c in range(E):
        nxt2 = jnp.where(nxt == c, nxt[0, c], nxt2)
    ordv = pick(ord_incl - nonz)
    first = ((jc == pick(t_excl)) & (jj == jc)).astype(jnp.int32)
    next_ex = pick(nxt)
    has_next = (next_ex < E).astype(jnp.int32)
    next_ex = jnp.minimum(next_ex, E - 1)
    next2_ex = pick(nxt2)
    has_next2 = (next2_ex < E).astype(jnp.int32)
    next2_ex = jnp.minimum(next2_ex, E - 1)
    tiles_ref[...] = jnp.concatenate(
        [blk, ex, start, end, ordv, first, next_ex, has_next,
         next2_ex, has_next2], axis=1)


def _routing(idx2):
    return pl.pallas_call(
        _routing_body,
        out_shape=(
            jax.ShapeDtypeStruct((T, 1), jnp.int32),
            jax.ShapeDtypeStruct((NUM_TILES, 10), jnp.int32),
        ),
    )(idx2)


@functools.cache
def _sc_mesh():
    return plsc.VectorSubcoreMesh(core_axis_name="c", subcore_axis_name="s")


def _dispatch(x, pos_w):
    @functools.partial(
        pl.kernel,
        mesh=_sc_mesh(),
        out_type=jax.ShapeDtypeStruct((T, H), jnp.float32),
        scratch_types=[
            pltpu.VMEM((BPW,), jnp.int32),
            pltpu.VMEM((BPW, H), jnp.float32),
            pltpu.SemaphoreType.DMA,
        ],
    )
    def k(x_hbm, pos_hbm, o_hbm, idx_v, rows_v, sem):
        wid = lax.axis_index("s") * NC + lax.axis_index("c")
        base = wid * BPW
        pltpu.sync_copy(pos_hbm.at[wid], idx_v)
        pltpu.sync_copy(x_hbm.at[pl.ds(base, BPW)], rows_v)
        pltpu.async_copy(rows_v, o_hbm.at[idx_v], sem).wait()

    return k(x, pos_w)


def _combine(y_sorted, pos_w):
    @functools.partial(
        pl.kernel,
        mesh=_sc_mesh(),
        out_type=jax.ShapeDtypeStruct((T, H), jnp.float32),
        scratch_types=[
            pltpu.VMEM((BPW,), jnp.int32),
            pltpu.VMEM((BPW, H), jnp.float32),
            pltpu.SemaphoreType.DMA,
        ],
    )
    def k(y_hbm, pos_hbm, o_hbm, idx_v, rows_v, sem):
        wid = lax.axis_index("s") * NC + lax.axis_index("c")
        base = wid * BPW
        pltpu.sync_copy(pos_hbm.at[wid], idx_v)
        pltpu.async_copy(y_hbm.at[idx_v], rows_v, sem).wait()
        pltpu.sync_copy(rows_v, o_hbm.at[pl.ds(base, BPW)])

    return k(y_sorted, pos_w)


def _moe_body(tiles_ref, x_ref, w1_hbm, b1_ref, w2_hbm, b2_ref, o_ref,
              w1_buf, w2_buf, s1, s2sem):
    j = pl.program_id(0)
    blk = tiles_ref[j, 0]
    ex = tiles_ref[j, 1]
    start = tiles_ref[j, 2]
    end = tiles_ref[j, 3]
    ordv = tiles_ref[j, 4]
    first = tiles_ref[j, 5]
    nxt = tiles_ref[j, 6]
    has_next = tiles_ref[j, 7]
    nxt2 = tiles_ref[j, 8]
    has_next2 = tiles_ref[j, 9]
    s3 = ordv % 3
    s2 = ordv % 2

    @pl.when(j == 0)
    def _():
        pltpu.make_async_copy(w1_hbm.at[ex], w1_buf.at[s3], s1).start()
        pltpu.make_async_copy(w2_hbm.at[ex], w2_buf.at[s2], s2sem).start()

        @pl.when(has_next == 1)
        def _():
            pltpu.make_async_copy(
                w1_hbm.at[nxt], w1_buf.at[(ordv + 1) % 3], s1).start()

    @pl.when(first == 1)
    def _():
        pltpu.make_async_copy(w1_hbm.at[ex], w1_buf.at[s3], s1).wait()
        pltpu.make_async_copy(w2_hbm.at[ex], w2_buf.at[s2], s2sem).wait()

        @pl.when(has_next2 == 1)
        def _():
            pltpu.make_async_copy(
                w1_hbm.at[nxt2], w1_buf.at[(ordv + 2) % 3], s1).start()

        @pl.when(has_next == 1)
        def _():
            pltpu.make_async_copy(
                w2_hbm.at[nxt], w2_buf.at[(ordv + 1) % 2], s2sem).start()

    h = jnp.dot(x_ref[...], w1_buf[s3], precision=lax.Precision.DEFAULT,
                preferred_element_type=jnp.float32)
    h = h + b1_ref[pl.ds(ex, 1), :]
    h = 0.5 * h * (1.0 + lax.erf(h * 0.7071067811865476))
    y = jnp.dot(h, w2_buf[s2], precision=lax.Precision.DEFAULT,
                preferred_element_type=jnp.float32)
    y = y + b2_ref[pl.ds(ex, 1), :]

    rows = blk * BM + lax.broadcasted_iota(jnp.int32, (BM, 1), 0)
    contrib = jnp.where((rows >= start) & (rows < end), y, 0.0)

    @pl.when(start == blk * BM)
    def _():
        o_ref[...] = contrib

    @pl.when(start != blk * BM)
    def _():
        o_ref[...] += contrib


def _moe(tiles, x_sorted, w1, b1, w2, b2):
    grid_spec = pltpu.PrefetchScalarGridSpec(
        num_scalar_prefetch=1,
        grid=(NUM_TILES,),
        in_specs=[
            pl.BlockSpec((BM, H), lambda j, t: (t[j, 0], 0)),
            pl.BlockSpec(memory_space=pl.ANY),
            pl.BlockSpec((E, F), lambda j, t: (0, 0)),
            pl.BlockSpec(memory_space=pl.ANY),
            pl.BlockSpec((E, H), lambda j, t: (0, 0)),
        ],
        out_specs=pl.BlockSpec((BM, H), lambda j, t: (t[j, 0], 0)),
        scratch_shapes=[
            pltpu.VMEM((3, H, F), jnp.float32),
            pltpu.VMEM((2, F, H), jnp.float32),
            pltpu.SemaphoreType.DMA,
            pltpu.SemaphoreType.DMA,
        ],
    )
    return pl.pallas_call(
        _moe_body,
        grid_spec=grid_spec,
        out_shape=jax.ShapeDtypeStruct((T, H), jnp.float32),
    )(tiles, x_sorted, w1, b1, w2, b2)


def kernel(x, expert_indices, W1, b1, W2, b2):
    idx2 = expert_indices.reshape(T, 1).astype(jnp.int32)
    pos, tiles = _routing(idx2)
    pos_w = pos.reshape(NW, BPW)
    x_sorted = _dispatch(x, pos_w)
    y_sorted = _moe(tiles, x_sorted, W1, b1, W2, b2)
    return _combine(y_sorted, pos_w)
